# baseline (speedup 1.0000x reference)
import dataclasses
import functools

import jax
import jax.numpy as jnp
from jax import lax
from jax.experimental import pallas as pl
from jax.experimental.pallas import tpu as pltpu
from jax.experimental.pallas import tpu_sc as plsc

NNZ = 1048576
D0, D1, D2 = 64, 4096, 256
SLICE_IDX = 10
OUT_N = D1 * D2

NUM_CORES = 2
NUM_SUBCORES = 16
NUM_TILES = NUM_CORES * NUM_SUBCORES
PER_TILE = NNZ // NUM_TILES
CHUNK = 4096
ROWS = CHUNK // 128
NCHUNK = PER_TILE // CHUNK
PER_SUB = OUT_N // NUM_SUBCORES
ZBUF = 4096

_mesh = plsc.VectorSubcoreMesh(core_axis_name="c", subcore_axis_name="s")

_cp = pltpu.CompilerParams()
if "needs_layout_passes" in pltpu.CompilerParams.__dataclass_fields__:
    _cp = dataclasses.replace(_cp, needs_layout_passes=False)


@functools.partial(
    pl.kernel,
    out_type=jax.ShapeDtypeStruct((2 * OUT_N,), jnp.float32),
    mesh=_mesh,
    compiler_params=_cp,
    scratch_types=[
        pltpu.VMEM((2, CHUNK), jnp.int32),
        pltpu.VMEM((2, CHUNK), jnp.int32),
        pltpu.VMEM((2, CHUNK), jnp.int32),
        pltpu.VMEM((2, CHUNK), jnp.float32),
        pltpu.VMEM((CHUNK + 128,), jnp.int32),
        pltpu.VMEM((128,), jnp.int32),
        pltpu.VMEM((128,), jnp.float32),
        pltpu.VMEM((ZBUF,), jnp.float32),
        pltpu.VMEM_SHARED((OUT_N,), jnp.float32),
        pltpu.SemaphoreType.DMA,
        pltpu.SemaphoreType.DMA,
        pltpu.SemaphoreType.DMA,
        pltpu.SemaphoreType.DMA,
    ],
)
def _sc_scatter(idx0_hbm, idx1_hbm, idx2_hbm, vals_hbm, out_hbm,
                b0, b1, b2, bv, pos_buf, lin_sbuf, val_sbuf, zbuf, accum,
                sem_a, sem_b, ssem_a, ssem_b):
    c = lax.axis_index("c")
    s = lax.axis_index("s")

    zero16 = jnp.zeros((16,), jnp.float32)

    @pl.loop(0, ZBUF, step=16)
    def _(i):
        zbuf[pl.ds(i, 16)] = zero16

    @pl.loop(0, PER_SUB, step=ZBUF)
    def _(k):
        pltpu.sync_copy(zbuf, accum.at[pl.ds(s * PER_SUB + k, ZBUF)])

    plsc.subcore_barrier()

    base = (c * NUM_SUBCORES + s) * PER_TILE
    in_sems = (sem_a, sem_b)
    iota16 = lax.iota(jnp.int32, 16)
    zeros16_i = jnp.zeros((16,), jnp.int32)
    zeros16_f = jnp.zeros((16,), jnp.float32)

    def issue_in(k, b):
        g = base + k * CHUNK
        return [
            pltpu.async_copy(idx0_hbm.at[pl.ds(g, CHUNK)], b0.at[b], in_sems[b]),
            pltpu.async_copy(idx1_hbm.at[pl.ds(g, CHUNK)], b1.at[b], in_sems[b]),
            pltpu.async_copy(idx2_hbm.at[pl.ds(g, CHUNK)], b2.at[b], in_sems[b]),
            pltpu.async_copy(vals_hbm.at[pl.ds(g, CHUNK)], bv.at[b], in_sems[b]),
        ]

    pending_in = {0: issue_in(0, 0)}
    for k in range(NCHUNK):
        b = k % 2
        for cp in pending_in.pop(k):
            cp.wait()
        if k + 1 < NCHUNK:
            pending_in[k + 1] = issue_in(k + 1, (k + 1) % 2)
        @plsc.parallel_loop(0, CHUNK, step=16, unroll=8,
                            carry=jnp.zeros((16,), jnp.int32))
        def off_v(co, off):
            m = b0[b, pl.ds(co, 16)] == SLICE_IDX
            cs = plsc.cumsum(m.astype(jnp.int32))
            cnt = plsc.all_reduce_population_count(m)
            pos = iota16 + co
            dst = off + cs - 1
            plsc.store_scatter(pos_buf, [dst], pos, mask=m)
            return off + cnt

        for j in range(8):
            plsc.store_scatter(pos_buf, [off_v + (iota16 + (j * 16))],
                               zeros16_i)

        cnt_sc = jnp.max(off_v)
        nblk = (cnt_sc + 127) >> 7

        @pl.loop(0, nblk)
        def _(j):
            o = j * 128
            for ii in range(8):
                lo = o + ii * 16
                pos16 = pos_buf[pl.ds(lo, 16)]
                bb = jnp.full((16,), b, jnp.int32)
                i1 = plsc.load_gather(b1, [bb, pos16])
                i2 = plsc.load_gather(b2, [bb, pos16])
                v = plsc.load_gather(bv, [bb, pos16])
                vmask = (iota16 + lo) < off_v
                val = jnp.where(vmask, v, 0.0)
                lin = (i1 << 8) | i2
                lin_sbuf[pl.ds(ii * 16, 16)] = lin
                val_sbuf[pl.ds(ii * 16, 16)] = val
            pltpu.sync_copy(val_sbuf, accum.at[lin_sbuf], add=True)

    plsc.subcore_barrier()

    pltpu.sync_copy(accum.at[pl.ds(s * PER_SUB, PER_SUB)],
                    out_hbm.at[pl.ds(c * OUT_N + s * PER_SUB, PER_SUB)])


def _combine_body(a_ref, b_ref, o_ref):
    s = a_ref[...] + b_ref[...]
    o_ref[...] = s.reshape(o_ref.shape)


def kernel(idx0, idx1, idx2, values):
    partials = _sc_scatter(idx0, idx1, idx2, values)
    nrow = D1 * D2 // 128
    p = partials.reshape(2 * nrow, 128)
    nblk = 16
    rb = nrow // nblk
    out = pl.pallas_call(
        _combine_body,
        grid=(nblk,),
        in_specs=[
            pl.BlockSpec((rb, 128), lambda i: (i, 0)),
            pl.BlockSpec((rb, 128), lambda i: (i + nblk, 0)),
        ],
        out_specs=pl.BlockSpec((D1 // nblk, D2), lambda i: (i, 0)),
        out_shape=jax.ShapeDtypeStruct((D1, D2), jnp.float32),
    )(p, p)
    return out.reshape(1, D1, D2)


# speedup vs baseline: 1.0330x; 1.0330x over previous
import dataclasses
import functools

import jax
import jax.numpy as jnp
from jax import lax
from jax.experimental import pallas as pl
from jax.experimental.pallas import tpu as pltpu
from jax.experimental.pallas import tpu_sc as plsc

NNZ = 1048576
D0, D1, D2 = 64, 4096, 256
SLICE_IDX = 10
OUT_N = D1 * D2

NUM_CORES = 2
NUM_SUBCORES = 16
NUM_TILES = NUM_CORES * NUM_SUBCORES
PER_TILE = NNZ // NUM_TILES
CHUNK = 16384
ROWS = CHUNK // 128
NCHUNK = PER_TILE // CHUNK
PER_SUB = OUT_N // NUM_SUBCORES
ZBUF = 4096

_mesh = plsc.VectorSubcoreMesh(core_axis_name="c", subcore_axis_name="s")

_cp = pltpu.CompilerParams()
if "needs_layout_passes" in pltpu.CompilerParams.__dataclass_fields__:
    _cp = dataclasses.replace(_cp, needs_layout_passes=False)


@functools.partial(
    pl.kernel,
    out_type=jax.ShapeDtypeStruct((2 * OUT_N,), jnp.float32),
    mesh=_mesh,
    compiler_params=_cp,
    scratch_types=[
        pltpu.VMEM((2, CHUNK), jnp.int32),
        pltpu.VMEM((CHUNK + 128,), jnp.int32),
        pltpu.VMEM((128,), jnp.int32),
        pltpu.VMEM((128,), jnp.int32),
        pltpu.VMEM((128,), jnp.int32),
        pltpu.VMEM((128,), jnp.float32),
        pltpu.VMEM((128,), jnp.int32),
        pltpu.VMEM((128,), jnp.float32),
        pltpu.VMEM((ZBUF,), jnp.float32),
        pltpu.VMEM_SHARED((OUT_N,), jnp.float32),
        pltpu.SemaphoreType.DMA,
        pltpu.SemaphoreType.DMA,
        pltpu.SemaphoreType.DMA,
        pltpu.SemaphoreType.DMA,
    ],
)
def _sc_scatter(idx0_hbm, idx1_hbm, idx2_hbm, vals_hbm, out_hbm,
                b0, pos_buf, gpos_buf, g1_buf, g2_buf, gv_buf,
                lin_sbuf, val_sbuf, zbuf, accum,
                sem_a, sem_b, ssem_a, ssem_b):
    c = lax.axis_index("c")
    s = lax.axis_index("s")

    zero16 = jnp.zeros((16,), jnp.float32)

    @pl.loop(0, ZBUF, step=16)
    def _(i):
        zbuf[pl.ds(i, 16)] = zero16

    @pl.loop(0, PER_SUB, step=ZBUF)
    def _(k):
        pltpu.sync_copy(zbuf, accum.at[pl.ds(s * PER_SUB + k, ZBUF)])

    plsc.subcore_barrier()

    base = (c * NUM_SUBCORES + s) * PER_TILE
    in_sems = (sem_a, sem_b)
    iota16 = lax.iota(jnp.int32, 16)
    zeros16_i = jnp.zeros((16,), jnp.int32)
    zeros16_f = jnp.zeros((16,), jnp.float32)

    def issue_in(k, b):
        g = base + k * CHUNK
        return [
            pltpu.async_copy(idx0_hbm.at[pl.ds(g, CHUNK)], b0.at[b], in_sems[b]),
        ]

    pending_in = {0: issue_in(0, 0)}
    for k in range(NCHUNK):
        b = k % 2
        for cp in pending_in.pop(k):
            cp.wait()
        if k + 1 < NCHUNK:
            pending_in[k + 1] = issue_in(k + 1, (k + 1) % 2)
        @plsc.parallel_loop(0, CHUNK, step=16, unroll=8,
                            carry=jnp.zeros((16,), jnp.int32))
        def off_v(co, off):
            m = b0[b, pl.ds(co, 16)] == SLICE_IDX
            cs = plsc.cumsum(m.astype(jnp.int32))
            cnt = plsc.all_reduce_population_count(m)
            pos = iota16 + co
            dst = off + cs - 1
            plsc.store_scatter(pos_buf, [dst], pos, mask=m)
            return off + cnt

        for j in range(8):
            plsc.store_scatter(pos_buf, [off_v + (iota16 + (j * 16))],
                               zeros16_i)

        cnt_sc = jnp.max(off_v)
        nblk = (cnt_sc + 127) >> 7

        gbase = base + k * CHUNK

        @pl.loop(0, nblk)
        def _(j):
            o = j * 128
            for ii in range(8):
                gpos_buf[pl.ds(ii * 16, 16)] = (
                    pos_buf[pl.ds(o + ii * 16, 16)] + gbase)
            h1 = pltpu.async_copy(idx1_hbm.at[gpos_buf], g1_buf, in_sems[b])
            h2 = pltpu.async_copy(idx2_hbm.at[gpos_buf], g2_buf, in_sems[b])
            h3 = pltpu.async_copy(vals_hbm.at[gpos_buf], gv_buf, in_sems[b])
            h1.wait()
            h2.wait()
            h3.wait()
            for ii in range(8):
                lo = o + ii * 16
                sl = pl.ds(ii * 16, 16)
                vmask = (iota16 + lo) < off_v
                val = jnp.where(vmask, gv_buf[sl], 0.0)
                lin = (g1_buf[sl] << 8) | g2_buf[sl]
                lin_sbuf[sl] = lin
                val_sbuf[sl] = val
            pltpu.sync_copy(val_sbuf, accum.at[lin_sbuf], add=True)

    plsc.subcore_barrier()

    pltpu.sync_copy(accum.at[pl.ds(s * PER_SUB, PER_SUB)],
                    out_hbm.at[pl.ds(c * OUT_N + s * PER_SUB, PER_SUB)])


def _combine_body(a_ref, b_ref, o_ref):
    s = a_ref[...] + b_ref[...]
    o_ref[...] = s.reshape(o_ref.shape)


def kernel(idx0, idx1, idx2, values):
    partials = _sc_scatter(idx0, idx1, idx2, values)
    nrow = D1 * D2 // 128
    p = partials.reshape(2 * nrow, 128)
    nblk = 16
    rb = nrow // nblk
    out = pl.pallas_call(
        _combine_body,
        grid=(nblk,),
        in_specs=[
            pl.BlockSpec((rb, 128), lambda i: (i, 0)),
            pl.BlockSpec((rb, 128), lambda i: (i + nblk, 0)),
        ],
        out_specs=pl.BlockSpec((D1 // nblk, D2), lambda i: (i, 0)),
        out_shape=jax.ShapeDtypeStruct((D1, D2), jnp.float32),
    )(p, p)
    return out.reshape(1, D1, D2)


# speedup vs baseline: 1.1007x; 1.0655x over previous
import dataclasses
import functools

import jax
import jax.numpy as jnp
from jax import lax
from jax.experimental import pallas as pl
from jax.experimental.pallas import tpu as pltpu
from jax.experimental.pallas import tpu_sc as plsc

NNZ = 1048576
D0, D1, D2 = 64, 4096, 256
SLICE_IDX = 10
OUT_N = D1 * D2

NUM_CORES = 2
NUM_SUBCORES = 16
NUM_TILES = NUM_CORES * NUM_SUBCORES
PER_TILE = NNZ // NUM_TILES
CHUNK = 16384
ROWS = CHUNK // 128
NCHUNK = PER_TILE // CHUNK
PER_SUB = OUT_N // NUM_SUBCORES
ZBUF = 4096

_mesh = plsc.VectorSubcoreMesh(core_axis_name="c", subcore_axis_name="s")

_cp = pltpu.CompilerParams()
if "needs_layout_passes" in pltpu.CompilerParams.__dataclass_fields__:
    _cp = dataclasses.replace(_cp, needs_layout_passes=False)


@functools.partial(
    pl.kernel,
    out_type=jax.ShapeDtypeStruct((2 * OUT_N,), jnp.float32),
    mesh=_mesh,
    compiler_params=_cp,
    scratch_types=[
        pltpu.VMEM((2, CHUNK), jnp.int32),
        pltpu.VMEM((CHUNK + 128,), jnp.int32),
        pltpu.VMEM((128,), jnp.int32),
        pltpu.VMEM((128,), jnp.int32),
        pltpu.VMEM((128,), jnp.int32),
        pltpu.VMEM((128,), jnp.float32),
        pltpu.VMEM((128,), jnp.int32),
        pltpu.VMEM((128,), jnp.float32),
        pltpu.VMEM((ZBUF,), jnp.float32),
        pltpu.VMEM_SHARED((OUT_N,), jnp.float32),
        pltpu.SemaphoreType.DMA,
        pltpu.SemaphoreType.DMA,
        pltpu.SemaphoreType.DMA,
        pltpu.SemaphoreType.DMA,
    ],
)
def _sc_scatter(idx0_hbm, idx1_hbm, idx2_hbm, vals_hbm, out_hbm,
                b0, pos_buf, gpos_buf, g1_buf, g2_buf, gv_buf,
                lin_sbuf, val_sbuf, zbuf, accum,
                sem_a, sem_b, ssem_a, ssem_b):
    c = lax.axis_index("c")
    s = lax.axis_index("s")

    zero16 = jnp.zeros((16,), jnp.float32)

    @pl.loop(0, ZBUF, step=16)
    def _(i):
        zbuf[pl.ds(i, 16)] = zero16

    @pl.loop(0, PER_SUB, step=ZBUF)
    def _(k):
        pltpu.sync_copy(zbuf, accum.at[pl.ds(s * PER_SUB + k, ZBUF)])

    plsc.subcore_barrier()

    base = (c * NUM_SUBCORES + s) * PER_TILE
    in_sems = (sem_a, sem_b)
    iota16 = lax.iota(jnp.int32, 16)
    zeros16_i = jnp.zeros((16,), jnp.int32)
    zeros16_f = jnp.zeros((16,), jnp.float32)

    def issue_in(k, b):
        g = base + k * CHUNK
        return [
            pltpu.async_copy(idx0_hbm.at[pl.ds(g, CHUNK)], b0.at[b], in_sems[b]),
        ]

    pending_in = {0: issue_in(0, 0)}
    for k in range(NCHUNK):
        b = k % 2
        for cp in pending_in.pop(k):
            cp.wait()
        if k + 1 < NCHUNK:
            pending_in[k + 1] = issue_in(k + 1, (k + 1) % 2)
        @plsc.parallel_loop(0, CHUNK, step=16, unroll=8,
                            carry=jnp.zeros((16,), jnp.int32))
        def off_v(co, off):
            m = b0[b, pl.ds(co, 16)] == SLICE_IDX
            cs = plsc.cumsum(m.astype(jnp.int32))
            cnt = plsc.all_reduce_population_count(m)
            pos = iota16 + co
            dst = off + cs - 1
            plsc.store_scatter(pos_buf, [dst], pos, mask=m)
            return off + cnt

        for j in range(8):
            plsc.store_scatter(pos_buf, [off_v + (iota16 + (j * 16))],
                               zeros16_i)

        cnt_sc = jnp.max(off_v)
        nblk = (cnt_sc + 127) >> 7

        gbase = base + k * CHUNK

        @pl.loop(0, nblk)
        def _(j):
            o = j * 128
            for ii in range(8):
                gpos_buf[pl.ds(ii * 16, 16)] = (
                    pos_buf[pl.ds(o + ii * 16, 16)] + gbase)
            h1 = pltpu.async_copy(idx1_hbm.at[gpos_buf], g1_buf, in_sems[b])
            h2 = pltpu.async_copy(idx2_hbm.at[gpos_buf], g2_buf, in_sems[b])
            h3 = pltpu.async_copy(vals_hbm.at[gpos_buf], gv_buf, in_sems[b])
            h1.wait()
            h2.wait()
            h3.wait()
            for ii in range(8):
                lo = o + ii * 16
                sl = pl.ds(ii * 16, 16)
                vmask = (iota16 + lo) < off_v
                val = jnp.where(vmask, gv_buf[sl], 0.0)
                lin = (g1_buf[sl] << 8) | g2_buf[sl]
                lin_sbuf[sl] = lin
                val_sbuf[sl] = val
            pltpu.sync_copy(val_sbuf, accum.at[lin_sbuf], add=True)

    plsc.subcore_barrier()

    pltpu.sync_copy(accum.at[pl.ds(s * PER_SUB, PER_SUB)],
                    out_hbm.at[pl.ds(c * OUT_N + s * PER_SUB, PER_SUB)])


def _combine_body(a_ref, b_ref, o_ref):
    s = a_ref[...] + b_ref[...]
    o_ref[...] = s.reshape(o_ref.shape)


def kernel(idx0, idx1, idx2, values):
    partials = _sc_scatter(idx0, idx1, idx2, values)
    nrow = D1 * D2 // 128
    p = partials.reshape(2 * nrow, 128)
    nblk = 8
    rb = nrow // nblk
    out = pl.pallas_call(
        _combine_body,
        grid=(nblk,),
        in_specs=[
            pl.BlockSpec((rb, 128), lambda i: (i, 0)),
            pl.BlockSpec((rb, 128), lambda i: (i + nblk, 0)),
        ],
        out_specs=pl.BlockSpec((D1 // nblk, D2), lambda i: (i, 0)),
        out_shape=jax.ShapeDtypeStruct((D1, D2), jnp.float32),
    )(p, p)
    return out.reshape(1, D1, D2)


# speedup vs baseline: 1.1530x; 1.0475x over previous
import dataclasses
import functools

import jax
import jax.numpy as jnp
from jax import lax
from jax.experimental import pallas as pl
from jax.experimental.pallas import tpu as pltpu
from jax.experimental.pallas import tpu_sc as plsc

NNZ = 1048576
D0, D1, D2 = 64, 4096, 256
SLICE_IDX = 10
OUT_N = D1 * D2

NUM_CORES = 2
NUM_SUBCORES = 16
NUM_TILES = NUM_CORES * NUM_SUBCORES
PER_TILE = NNZ // NUM_TILES
CHUNK = 16384
ROWS = CHUNK // 128
NCHUNK = PER_TILE // CHUNK
PER_SUB = OUT_N // NUM_SUBCORES
ZBUF = 4096

_mesh = plsc.VectorSubcoreMesh(core_axis_name="c", subcore_axis_name="s")

_cp = pltpu.CompilerParams()
if "needs_layout_passes" in pltpu.CompilerParams.__dataclass_fields__:
    _cp = dataclasses.replace(_cp, needs_layout_passes=False)


@functools.partial(
    pl.kernel,
    out_type=jax.ShapeDtypeStruct((2 * OUT_N,), jnp.float32),
    mesh=_mesh,
    compiler_params=_cp,
    scratch_types=[
        pltpu.VMEM((2, CHUNK), jnp.int32),
        pltpu.VMEM((CHUNK + 128,), jnp.int32),
        pltpu.VMEM((128,), jnp.int32),
        pltpu.VMEM((128,), jnp.int32),
        pltpu.VMEM((128,), jnp.int32),
        pltpu.VMEM((128,), jnp.float32),
        pltpu.VMEM((128,), jnp.int32),
        pltpu.VMEM((128,), jnp.float32),
        pltpu.VMEM((ZBUF,), jnp.float32),
        pltpu.VMEM_SHARED((OUT_N,), jnp.float32),
        pltpu.SemaphoreType.DMA,
        pltpu.SemaphoreType.DMA,
        pltpu.SemaphoreType.DMA,
        pltpu.SemaphoreType.DMA,
    ],
)
def _sc_scatter(idx0_hbm, idx1_hbm, idx2_hbm, vals_hbm, out_hbm,
                b0, pos_buf, gpos_buf, g1_buf, g2_buf, gv_buf,
                lin_sbuf, val_sbuf, zbuf, accum,
                sem_a, sem_b, ssem_a, ssem_b):
    c = lax.axis_index("c")
    s = lax.axis_index("s")

    zero16 = jnp.zeros((16,), jnp.float32)

    @pl.loop(0, ZBUF, step=16)
    def _(i):
        zbuf[pl.ds(i, 16)] = zero16

    @pl.loop(0, PER_SUB, step=ZBUF)
    def _(k):
        pltpu.sync_copy(zbuf, accum.at[pl.ds(s * PER_SUB + k, ZBUF)])

    plsc.subcore_barrier()

    base = (c * NUM_SUBCORES + s) * PER_TILE
    in_sems = (sem_a, sem_b)
    iota16 = lax.iota(jnp.int32, 16)
    zeros16_i = jnp.zeros((16,), jnp.int32)
    zeros16_f = jnp.zeros((16,), jnp.float32)

    def issue_in(k, b):
        g = base + k * CHUNK
        return [
            pltpu.async_copy(idx0_hbm.at[pl.ds(g, CHUNK)], b0.at[b], in_sems[b]),
        ]

    pending_in = {0: issue_in(0, 0)}
    for k in range(NCHUNK):
        b = k % 2
        for cp in pending_in.pop(k):
            cp.wait()
        if k + 1 < NCHUNK:
            pending_in[k + 1] = issue_in(k + 1, (k + 1) % 2)
        @plsc.parallel_loop(0, CHUNK, step=16, unroll=8,
                            carry=jnp.zeros((16,), jnp.int32))
        def off_v(co, off):
            m = b0[b, pl.ds(co, 16)] == SLICE_IDX
            cs = plsc.cumsum(m.astype(jnp.int32))
            cnt = plsc.all_reduce_population_count(m)
            pos = iota16 + co
            dst = off + cs - 1
            plsc.store_scatter(pos_buf, [dst], pos, mask=m)
            return off + cnt

        for j in range(8):
            plsc.store_scatter(pos_buf, [off_v + (iota16 + (j * 16))],
                               zeros16_i)

        cnt_sc = jnp.max(off_v)
        nblk = (cnt_sc + 127) >> 7

        gbase = base + k * CHUNK

        @pl.loop(0, nblk)
        def _(j):
            o = j * 128
            for ii in range(8):
                gpos_buf[pl.ds(ii * 16, 16)] = (
                    pos_buf[pl.ds(o + ii * 16, 16)] + gbase)
            h1 = pltpu.async_copy(idx1_hbm.at[gpos_buf], g1_buf, in_sems[b])
            h2 = pltpu.async_copy(idx2_hbm.at[gpos_buf], g2_buf, in_sems[b])
            h3 = pltpu.async_copy(vals_hbm.at[gpos_buf], gv_buf, in_sems[b])
            h1.wait()
            h2.wait()
            h3.wait()
            for ii in range(8):
                lo = o + ii * 16
                sl = pl.ds(ii * 16, 16)
                vmask = (iota16 + lo) < off_v
                val = jnp.where(vmask, gv_buf[sl], 0.0)
                lin = (g1_buf[sl] << 8) | g2_buf[sl]
                lin_sbuf[sl] = lin
                val_sbuf[sl] = val
            pltpu.sync_copy(val_sbuf, accum.at[lin_sbuf], add=True)

    plsc.subcore_barrier()

    pltpu.sync_copy(accum.at[pl.ds(s * PER_SUB, PER_SUB)],
                    out_hbm.at[pl.ds(c * OUT_N + s * PER_SUB, PER_SUB)])


def _combine_body(a_ref, b_ref, o_ref):
    s = a_ref[...] + b_ref[...]
    o_ref[...] = s.reshape(o_ref.shape)


def kernel(idx0, idx1, idx2, values):
    partials = _sc_scatter(idx0, idx1, idx2, values)
    nrow = D1 * D2 // 128
    p = partials.reshape(2 * nrow, 128)
    nblk = 4
    rb = nrow // nblk
    out = pl.pallas_call(
        _combine_body,
        grid=(nblk,),
        in_specs=[
            pl.BlockSpec((rb, 128), lambda i: (i, 0)),
            pl.BlockSpec((rb, 128), lambda i: (i + nblk, 0)),
        ],
        out_specs=pl.BlockSpec((D1 // nblk, D2), lambda i: (i, 0)),
        out_shape=jax.ShapeDtypeStruct((D1, D2), jnp.float32),
    )(p, p)
    return out.reshape(1, D1, D2)


# speedup vs baseline: 1.1733x; 1.0176x over previous
import dataclasses
import functools

import jax
import jax.numpy as jnp
from jax import lax
from jax.experimental import pallas as pl
from jax.experimental.pallas import tpu as pltpu
from jax.experimental.pallas import tpu_sc as plsc

NNZ = 1048576
D0, D1, D2 = 64, 4096, 256
SLICE_IDX = 10
OUT_N = D1 * D2

NUM_CORES = 2
NUM_SUBCORES = 16
NUM_TILES = NUM_CORES * NUM_SUBCORES
PER_TILE = NNZ // NUM_TILES
CHUNK = 16384
ROWS = CHUNK // 128
NCHUNK = PER_TILE // CHUNK
PER_SUB = OUT_N // NUM_SUBCORES
ZBUF = 4096

_mesh = plsc.VectorSubcoreMesh(core_axis_name="c", subcore_axis_name="s")

_cp = pltpu.CompilerParams()
if "needs_layout_passes" in pltpu.CompilerParams.__dataclass_fields__:
    _cp = dataclasses.replace(_cp, needs_layout_passes=False)


@functools.partial(
    pl.kernel,
    out_type=jax.ShapeDtypeStruct((2 * OUT_N,), jnp.float32),
    mesh=_mesh,
    compiler_params=_cp,
    scratch_types=[
        pltpu.VMEM((2, CHUNK), jnp.int32),
        pltpu.VMEM((CHUNK + 128,), jnp.int32),
        pltpu.VMEM((128,), jnp.int32),
        pltpu.VMEM((128,), jnp.int32),
        pltpu.VMEM((128,), jnp.int32),
        pltpu.VMEM((128,), jnp.float32),
        pltpu.VMEM((128,), jnp.int32),
        pltpu.VMEM((128,), jnp.float32),
        pltpu.VMEM((ZBUF,), jnp.float32),
        pltpu.VMEM_SHARED((OUT_N,), jnp.float32),
        pltpu.SemaphoreType.DMA,
        pltpu.SemaphoreType.DMA,
        pltpu.SemaphoreType.DMA,
        pltpu.SemaphoreType.DMA,
    ],
)
def _sc_scatter(idx0_hbm, idx1_hbm, idx2_hbm, vals_hbm, out_hbm,
                b0, pos_buf, gpos_buf, g1_buf, g2_buf, gv_buf,
                lin_sbuf, val_sbuf, zbuf, accum,
                sem_a, sem_b, ssem_a, ssem_b):
    c = lax.axis_index("c")
    s = lax.axis_index("s")

    zero16 = jnp.zeros((16,), jnp.float32)

    @pl.loop(0, ZBUF, step=16)
    def _(i):
        zbuf[pl.ds(i, 16)] = zero16

    @pl.loop(0, PER_SUB, step=ZBUF)
    def _(k):
        pltpu.sync_copy(zbuf, accum.at[pl.ds(s * PER_SUB + k, ZBUF)])

    plsc.subcore_barrier()

    base = (c * NUM_SUBCORES + s) * PER_TILE
    in_sems = (sem_a, sem_b)
    iota16 = lax.iota(jnp.int32, 16)
    zeros16_i = jnp.zeros((16,), jnp.int32)
    zeros16_f = jnp.zeros((16,), jnp.float32)

    def issue_in(k, b):
        g = base + k * CHUNK
        return [
            pltpu.async_copy(idx0_hbm.at[pl.ds(g, CHUNK)], b0.at[b], in_sems[b]),
        ]

    pending_in = {0: issue_in(0, 0)}
    for k in range(NCHUNK):
        b = k % 2
        for cp in pending_in.pop(k):
            cp.wait()
        if k + 1 < NCHUNK:
            pending_in[k + 1] = issue_in(k + 1, (k + 1) % 2)
        @plsc.parallel_loop(0, CHUNK, step=16, unroll=8,
                            carry=jnp.zeros((16,), jnp.int32))
        def off_v(co, off):
            m = b0[b, pl.ds(co, 16)] == SLICE_IDX
            cs = plsc.cumsum(m.astype(jnp.int32))
            cnt = plsc.all_reduce_population_count(m)
            pos = iota16 + co
            dst = off + cs - 1
            plsc.store_scatter(pos_buf, [dst], pos, mask=m)
            return off + cnt

        for j in range(8):
            plsc.store_scatter(pos_buf, [off_v + (iota16 + (j * 16))],
                               zeros16_i)

        cnt_sc = jnp.max(off_v)
        nblk = (cnt_sc + 127) >> 7

        gbase = base + k * CHUNK

        @pl.loop(0, nblk)
        def _(j):
            o = j * 128
            for ii in range(8):
                gpos_buf[pl.ds(ii * 16, 16)] = (
                    pos_buf[pl.ds(o + ii * 16, 16)] + gbase)
            h1 = pltpu.async_copy(idx1_hbm.at[gpos_buf], g1_buf, in_sems[b])
            h2 = pltpu.async_copy(idx2_hbm.at[gpos_buf], g2_buf, in_sems[b])
            h3 = pltpu.async_copy(vals_hbm.at[gpos_buf], gv_buf, in_sems[b])
            h1.wait()
            h2.wait()
            h3.wait()
            for ii in range(8):
                lo = o + ii * 16
                sl = pl.ds(ii * 16, 16)
                vmask = (iota16 + lo) < off_v
                val = jnp.where(vmask, gv_buf[sl], 0.0)
                lin = (g1_buf[sl] << 8) | g2_buf[sl]
                lin_sbuf[sl] = lin
                val_sbuf[sl] = val
            pltpu.sync_copy(val_sbuf, accum.at[lin_sbuf], add=True)

    plsc.subcore_barrier()

    pltpu.sync_copy(accum.at[pl.ds(s * PER_SUB, PER_SUB)],
                    out_hbm.at[pl.ds(c * OUT_N + s * PER_SUB, PER_SUB)])


def _combine_body(a_ref, b_ref, o_ref):
    s = a_ref[...] + b_ref[...]
    o_ref[...] = s.reshape(o_ref.shape)


def kernel(idx0, idx1, idx2, values):
    partials = _sc_scatter(idx0, idx1, idx2, values)
    nrow = D1 * D2 // 128
    p = partials.reshape(2 * nrow, 128)
    nblk = 2
    rb = nrow // nblk
    out = pl.pallas_call(
        _combine_body,
        grid=(nblk,),
        in_specs=[
            pl.BlockSpec((rb, 128), lambda i: (i, 0)),
            pl.BlockSpec((rb, 128), lambda i: (i + nblk, 0)),
        ],
        out_specs=pl.BlockSpec((D1 // nblk, D2), lambda i: (i, 0)),
        out_shape=jax.ShapeDtypeStruct((D1, D2), jnp.float32),
    )(p, p)
    return out.reshape(1, D1, D2)
